# baseline (speedup 1.0000x reference)
import functools

import jax
import jax.numpy as jnp
from jax import lax
from jax.experimental import pallas as pl
from jax.experimental.pallas import tpu as pltpu
from jax.experimental.pallas import tpu_sc as plsc

D = 64
EPS = 1e-5
L = 16
NC = 2
NS = 16
NW = NC * NS
BB = 128


def _rsqrt_scalar(x):
    i = lax.bitcast_convert_type(x, jnp.int32)
    i = jnp.int32(0x5F3759DF) - lax.shift_right_logical(i, 1)
    y = lax.bitcast_convert_type(i, jnp.float32)
    for _ in range(2):
        y = y * (1.5 - 0.5 * x * y * y)
    return y


@functools.partial(jax.jit, static_argnames=("n_tok", "n_b"))
def _sc_embed_ln(xt_flat, table, gamma, beta, *, n_tok, n_b):
    n_rows = n_tok * n_b
    u_per_w = (n_rows // BB) // NW
    mesh = plsc.VectorSubcoreMesh(core_axis_name="c", subcore_axis_name="s")

    @functools.partial(
        pl.kernel,
        mesh=mesh,
        compiler_params=pltpu.CompilerParams(
            needs_layout_passes=False, use_tc_tiling_on_sc=False),
        out_type=jax.ShapeDtypeStruct((n_rows, D), jnp.float32),
        scratch_types=[
            pltpu.VMEM((u_per_w * BB,), jnp.int32),
            pltpu.VMEM((BB, D), jnp.float32),
            pltpu.VMEM((BB, D), jnp.float32),
            pltpu.VMEM((D,), jnp.float32),
            pltpu.VMEM((D,), jnp.float32),
            pltpu.SemaphoreType.DMA,
            pltpu.SemaphoreType.DMA,
            pltpu.SemaphoreType.DMA,
            pltpu.SemaphoreType.DMA,
        ],
    )
    def body(x_hbm, table_hbm, gam_hbm, bet_hbm, out_hbm,
             idx_all, rows0, rows1, gam_v, bet_v,
             sg0, sg1, so0, so1):
        wid = lax.axis_index("s") * NC + lax.axis_index("c")
        u0 = wid * u_per_w
        rows = (rows0, rows1)
        sgs = (sg0, sg1)
        sos = (so0, so1)

        pltpu.sync_copy(x_hbm.at[pl.ds(u0 * BB, u_per_w * BB)], idx_all)
        pltpu.sync_copy(gam_hbm, gam_v)
        pltpu.sync_copy(bet_hbm, bet_v)
        g_vecs = [gam_v[pl.ds(16 * k, L)] for k in range(D // L)]
        b_vecs = [bet_v[pl.ds(16 * k, L)] for k in range(D // L)]

        def gather(u_local, b):
            pltpu.async_copy(
                table_hbm.at[idx_all.at[pl.ds(u_local * BB, BB)]],
                rows[b], sgs[b])

        def wait_gather(b):
            pltpu.make_async_copy(
                table_hbm.at[idx_all.at[pl.ds(0, BB)]], rows[b],
                sgs[b]).wait()

        def writeout(u_local, b):
            pltpu.async_copy(
                rows[b], out_hbm.at[pl.ds((u0 + u_local) * BB, BB)], sos[b])

        def wait_writeout(b):
            pltpu.make_async_copy(
                rows[b], out_hbm.at[pl.ds(0, BB)], sos[b]).wait()

        def compute(b):
            rows_v = rows[b]

            @plsc.parallel_loop(0, BB, unroll=8)
            def _(r):
                row = rows_v.at[r]
                a = [row[pl.ds(16 * k, L)] for k in range(D // L)]
                s4 = (a[0] + a[1]) + (a[2] + a[3])
                q4 = (a[0] * a[0] + a[1] * a[1]) + (a[2] * a[2] + a[3] * a[3])
                ssum = jnp.sum(s4)
                qsum = jnp.sum(q4)
                mean = ssum * (1.0 / D)
                var = qsum * (1.0 / D) - mean * mean
                rstd = _rsqrt_scalar(var + (EPS / D))
                p = mean * rstd
                for k in range(D // L):
                    row[pl.ds(16 * k, L)] = (
                        (a[k] * rstd - p) * g_vecs[k] + b_vecs[k])

        gather(0, 0)

        def unit_step(i, carry):
            for b in range(2):
                u_local = 2 * i + b

                @pl.when(jnp.logical_and(u_local >= 1,
                                         u_local + 1 < u_per_w))
                def _():
                    wait_writeout(1 - b)

                @pl.when(u_local + 1 < u_per_w)
                def _():
                    gather(u_local + 1, 1 - b)

                wait_gather(b)
                compute(b)
                writeout(u_local, b)
            return carry

        lax.fori_loop(0, u_per_w // 2, unit_step, 0)
        wait_writeout(0)
        wait_writeout(1)

    return body(xt_flat, table, gamma, beta)


def kernel(x, table, gamma, beta):
    n_b, n_tok = x.shape
    xt_flat = x.T.reshape(n_b * n_tok).astype(jnp.int32)
    y = _sc_embed_ln(xt_flat, table, gamma, beta, n_tok=n_tok, n_b=n_b)
    return y.reshape(n_tok, n_b, D).transpose((1, 0, 2))


# speedup vs baseline: 1.0108x; 1.0108x over previous
import functools

import jax
import jax.numpy as jnp
from jax import lax
from jax.experimental import pallas as pl
from jax.experimental.pallas import tpu as pltpu
from jax.experimental.pallas import tpu_sc as plsc

D = 64
EPS = 1e-5
L = 16
NC = 2
NS = 16
NW = NC * NS
BB = 128


def _rsqrt_scalar(x):
    i = lax.bitcast_convert_type(x, jnp.int32)
    i = jnp.int32(0x5F3759DF) - lax.shift_right_logical(i, 1)
    y = lax.bitcast_convert_type(i, jnp.float32)
    for _ in range(2):
        y = y * (1.5 - 0.5 * x * y * y)
    return y


@functools.partial(jax.jit, static_argnames=("n_tok", "n_b"))
def _sc_embed_ln(xt_flat, table, gamma, beta, *, n_tok, n_b):
    n_rows = n_tok * n_b
    u_per_w = (n_rows // BB) // NW
    mesh = plsc.VectorSubcoreMesh(core_axis_name="c", subcore_axis_name="s")

    @functools.partial(
        pl.kernel,
        mesh=mesh,
        compiler_params=pltpu.CompilerParams(
            needs_layout_passes=False, use_tc_tiling_on_sc=False),
        out_type=jax.ShapeDtypeStruct((n_rows, D), jnp.float32),
        scratch_types=[
            pltpu.VMEM((u_per_w * BB,), jnp.int32),
            pltpu.VMEM((BB, D), jnp.float32),
            pltpu.VMEM((BB, D), jnp.float32),
            pltpu.VMEM((D,), jnp.float32),
            pltpu.VMEM((D,), jnp.float32),
            pltpu.SemaphoreType.DMA,
            pltpu.SemaphoreType.DMA,
            pltpu.SemaphoreType.DMA,
            pltpu.SemaphoreType.DMA,
        ],
    )
    def body(x_hbm, table_hbm, gam_hbm, bet_hbm, out_hbm,
             idx_all, rows0, rows1, gam_v, bet_v,
             sg0, sg1, so0, so1):
        wid = lax.axis_index("s") * NC + lax.axis_index("c")
        u0 = wid * u_per_w
        rows = (rows0, rows1)
        sgs = (sg0, sg1)
        sos = (so0, so1)

        pltpu.sync_copy(x_hbm.at[pl.ds(u0 * BB, u_per_w * BB)], idx_all)
        pltpu.sync_copy(gam_hbm, gam_v)
        pltpu.sync_copy(bet_hbm, bet_v)
        g_vecs = [gam_v[pl.ds(16 * k, L)] for k in range(D // L)]
        b_vecs = [bet_v[pl.ds(16 * k, L)] for k in range(D // L)]

        def gather(u_local, b):
            pltpu.async_copy(
                table_hbm.at[idx_all.at[pl.ds(u_local * BB, BB)]],
                rows[b], sgs[b])

        def wait_gather(b):
            pltpu.make_async_copy(
                table_hbm.at[idx_all.at[pl.ds(0, BB)]], rows[b],
                sgs[b]).wait()

        def writeout(u_local, b):
            pltpu.async_copy(
                rows[b], out_hbm.at[pl.ds((u0 + u_local) * BB, BB)], sos[b])

        def wait_writeout(b):
            pltpu.make_async_copy(
                rows[b], out_hbm.at[pl.ds(0, BB)], sos[b]).wait()

        def compute(b):
            rows_v = rows[b]

            @plsc.parallel_loop(0, BB, unroll=4)
            def _(r):
                row = rows_v.at[r]
                a = [row[pl.ds(16 * k, L)] for k in range(D // L)]
                s4 = (a[0] + a[1]) + (a[2] + a[3])
                q4 = (a[0] * a[0] + a[1] * a[1]) + (a[2] * a[2] + a[3] * a[3])
                ssum = jnp.sum(s4)
                qsum = jnp.sum(q4)
                mean = ssum * (1.0 / D)
                var = qsum * (1.0 / D) - mean * mean
                rstd = _rsqrt_scalar(var + (EPS / D))
                p = mean * rstd
                for k in range(D // L):
                    row[pl.ds(16 * k, L)] = (
                        (a[k] * rstd - p) * g_vecs[k] + b_vecs[k])

        gather(0, 0)

        def unit_step(i, carry):
            for b in range(2):
                u_local = 2 * i + b

                @pl.when(jnp.logical_and(u_local >= 1,
                                         u_local + 1 < u_per_w))
                def _():
                    wait_writeout(1 - b)

                @pl.when(u_local + 1 < u_per_w)
                def _():
                    gather(u_local + 1, 1 - b)

                wait_gather(b)
                compute(b)
                writeout(u_local, b)
            return carry

        lax.fori_loop(0, u_per_w // 2, unit_step, 0)
        wait_writeout(0)
        wait_writeout(1)

    return body(xt_flat, table, gamma, beta)


def kernel(x, table, gamma, beta):
    n_b, n_tok = x.shape
    xt_flat = x.T.reshape(n_b * n_tok).astype(jnp.int32)
    y = _sc_embed_ln(xt_flat, table, gamma, beta, n_tok=n_tok, n_b=n_b)
    return y.reshape(n_tok, n_b, D).transpose((1, 0, 2))


# speedup vs baseline: 1.0237x; 1.0127x over previous
import functools

import jax
import jax.numpy as jnp
from jax import lax
from jax.experimental import pallas as pl
from jax.experimental.pallas import tpu as pltpu
from jax.experimental.pallas import tpu_sc as plsc

D = 64
EPS = 1e-5
L = 16
NC = 2
NS = 16
NW = NC * NS
BB = 128


def _rsqrt_scalar(x):
    i = lax.bitcast_convert_type(x, jnp.int32)
    i = jnp.int32(0x5F3759DF) - lax.shift_right_logical(i, 1)
    y = lax.bitcast_convert_type(i, jnp.float32)
    for _ in range(2):
        y = y * (1.5 - 0.5 * x * y * y)
    return y


@functools.partial(jax.jit, static_argnames=("n_tok", "n_b"))
def _sc_embed_ln(xt_flat, table, gamma, beta, *, n_tok, n_b):
    n_rows = n_tok * n_b
    u_per_w = (n_rows // BB) // NW
    mesh = plsc.VectorSubcoreMesh(core_axis_name="c", subcore_axis_name="s")

    @functools.partial(
        pl.kernel,
        mesh=mesh,
        compiler_params=pltpu.CompilerParams(
            needs_layout_passes=False, use_tc_tiling_on_sc=False),
        out_type=jax.ShapeDtypeStruct((n_rows, D), jnp.float32),
        scratch_types=[
            pltpu.VMEM((u_per_w * BB,), jnp.int32),
            pltpu.VMEM((BB, D), jnp.float32),
            pltpu.VMEM((BB, D), jnp.float32),
            pltpu.VMEM((D,), jnp.float32),
            pltpu.VMEM((D,), jnp.float32),
            pltpu.SemaphoreType.DMA,
            pltpu.SemaphoreType.DMA,
            pltpu.SemaphoreType.DMA,
            pltpu.SemaphoreType.DMA,
        ],
    )
    def body(x_hbm, table_hbm, gam_hbm, bet_hbm, out_hbm,
             idx_all, rows0, rows1, gam_v, bet_v,
             sg0, sg1, so0, so1):
        wid = lax.axis_index("s") * NC + lax.axis_index("c")
        u0 = wid * u_per_w
        rows = (rows0, rows1)
        sgs = (sg0, sg1)
        sos = (so0, so1)

        pltpu.sync_copy(x_hbm.at[pl.ds(u0 * BB, u_per_w * BB)], idx_all)
        pltpu.sync_copy(gam_hbm, gam_v)
        pltpu.sync_copy(bet_hbm, bet_v)
        g_vecs = [gam_v[pl.ds(16 * k, L)] for k in range(D // L)]
        b_vecs = [bet_v[pl.ds(16 * k, L)] for k in range(D // L)]

        def gather(u_local, b):
            pltpu.async_copy(
                table_hbm.at[idx_all.at[pl.ds(u_local * BB, BB)]],
                rows[b], sgs[b])

        def wait_gather(b):
            pltpu.make_async_copy(
                table_hbm.at[idx_all.at[pl.ds(0, BB)]], rows[b],
                sgs[b]).wait()

        def writeout(u_local, b):
            pltpu.async_copy(
                rows[b], out_hbm.at[pl.ds((u0 + u_local) * BB, BB)], sos[b])

        def wait_writeout(b):
            pltpu.make_async_copy(
                rows[b], out_hbm.at[pl.ds(0, BB)], sos[b]).wait()

        def compute(b):
            rows_v = rows[b]

            @plsc.parallel_loop(0, BB, unroll=2)
            def _(r):
                row = rows_v.at[r]
                a = [row[pl.ds(16 * k, L)] for k in range(D // L)]
                s4 = (a[0] + a[1]) + (a[2] + a[3])
                q4 = (a[0] * a[0] + a[1] * a[1]) + (a[2] * a[2] + a[3] * a[3])
                ssum = jnp.sum(s4)
                qsum = jnp.sum(q4)
                mean = ssum * (1.0 / D)
                var = qsum * (1.0 / D) - mean * mean
                rstd = _rsqrt_scalar(var + (EPS / D))
                p = mean * rstd
                for k in range(D // L):
                    row[pl.ds(16 * k, L)] = (
                        (a[k] * rstd - p) * g_vecs[k] + b_vecs[k])

        gather(0, 0)

        def unit_step(i, carry):
            for b in range(2):
                u_local = 2 * i + b

                @pl.when(jnp.logical_and(u_local >= 1,
                                         u_local + 1 < u_per_w))
                def _():
                    wait_writeout(1 - b)

                @pl.when(u_local + 1 < u_per_w)
                def _():
                    gather(u_local + 1, 1 - b)

                wait_gather(b)
                compute(b)
                writeout(u_local, b)
            return carry

        lax.fori_loop(0, u_per_w // 2, unit_step, 0)
        wait_writeout(0)
        wait_writeout(1)

    return body(xt_flat, table, gamma, beta)


def kernel(x, table, gamma, beta):
    n_b, n_tok = x.shape
    xt_flat = x.T.reshape(n_b * n_tok).astype(jnp.int32)
    y = _sc_embed_ln(xt_flat, table, gamma, beta, n_tok=n_tok, n_b=n_b)
    return y.reshape(n_tok, n_b, D).transpose((1, 0, 2))
